# Initial kernel scaffold; baseline (speedup 1.0000x reference)
import jax
import jax.numpy as jnp
from jax.experimental import pallas as pl


def kernel(class_logits, bbox_deltas, roi_proposals, resized_image_sizes):
    raise NotImplementedError("write your pallas kernel here")


# speedup vs baseline: 649.5809x; 649.5809x over previous
import functools

import jax
import jax.numpy as jnp
from jax.experimental import pallas as pl

SCORE_THRESH = 0.05
NMS_THRESH = 0.5
DETS_PER_IMG = 100
BBOX_XFORM_CLIP = 4.135166556742356

N_PAD = 5120


def _stage1_kernel(logits_ref, deltas_ref, props_ref, size_ref,
                   probs_ref, labels_ref, x1_ref, y1_ref, x2_ref, y2_ref):
    l = logits_ref[0]
    n, c = l.shape
    m = jnp.max(l, axis=1, keepdims=True)
    e = jnp.exp(l - m)
    s = jnp.sum(e, axis=1, keepdims=True)
    probs = 1.0 / s
    cio = jax.lax.broadcasted_iota(jnp.int32, (n, c), 1).astype(jnp.float32)
    labels = jnp.min(jnp.where(e == 1.0, cio, float(c)), axis=1,
                     keepdims=True)

    d = deltas_ref[0]
    dio = jax.lax.broadcasted_iota(jnp.int32, (n, 4 * c), 1).astype(jnp.float32)
    base = 4.0 * labels
    dm = [jnp.sum(jnp.where(dio == base + float(j), d, 0.0), axis=1,
                  keepdims=True) for j in range(4)]
    dx, dy, dw, dh = dm

    p = props_ref[0]
    px1 = p[:, 0:1]
    py1 = p[:, 1:2]
    px2 = p[:, 2:3]
    py2 = p[:, 3:4]
    widths = px2 - px1
    heights = py2 - py1
    ctr_x = px1 + 0.5 * widths
    ctr_y = py1 + 0.5 * heights
    dw = jnp.minimum(dw, BBOX_XFORM_CLIP)
    dh = jnp.minimum(dh, BBOX_XFORM_CLIP)
    pred_ctr_x = dx * widths + ctr_x
    pred_ctr_y = dy * heights + ctr_y
    pred_w = jnp.exp(dw) * widths
    pred_h = jnp.exp(dh) * heights
    x1 = pred_ctr_x - 0.5 * pred_w
    y1 = pred_ctr_y - 0.5 * pred_h
    x2 = pred_ctr_x + 0.5 * pred_w
    y2 = pred_ctr_y + 0.5 * pred_h

    sz = size_ref[0].astype(jnp.float32)
    h_img = sz[0:1, 0:1]
    w_img = sz[0:1, 1:2]
    x1 = jnp.clip(x1, 0.0, w_img)
    y1 = jnp.clip(y1, 0.0, h_img)
    x2 = jnp.clip(x2, 0.0, w_img)
    y2 = jnp.clip(y2, 0.0, h_img)

    probs_ref[0] = probs
    labels_ref[0] = labels
    x1_ref[0] = x1
    y1_ref[0] = y1
    x2_ref[0] = x2
    y2_ref[0] = y2


def _stage2_kernel(nimg, probs_ref, labels_ref, x1_ref, y1_ref, x2_ref,
                   y2_ref, boxes_ref, scores_ref, labout_ref):
    probs = probs_ref[...]
    labels = labels_ref[...]
    x1 = x1_ref[...]
    y1 = y1_ref[...]
    x2 = x2_ref[...]
    y2 = y2_ref[...]

    valid = (labels > 0.0) & (probs > SCORE_THRESH)
    sc0 = jnp.where(valid, probs, -1.0)

    r = 40
    lane128 = jax.lax.broadcasted_iota(jnp.int32, (8, 128), 1).astype(
        jnp.float32)
    row8 = jax.lax.broadcasted_iota(jnp.int32, (8, 128), 0).astype(
        jnp.float32)
    idxf = (jax.lax.broadcasted_iota(jnp.int32, (r, 128), 0) * 128
            + jax.lax.broadcasted_iota(jnp.int32, (r, 128), 1)).astype(
        jnp.float32)

    sls = [slice(128 * b, 128 * (b + 1)) for b in range(nimg)]
    maxcs = []
    offs = []
    for b in range(nimg):
        sl = sls[b]
        mc = jnp.maximum(
            jnp.maximum(jnp.max(x1[:, sl]), jnp.max(y1[:, sl])),
            jnp.maximum(jnp.max(x2[:, sl]), jnp.max(y2[:, sl])))
        maxcs.append(mc)
        offs.append(labels[:, sl] * (mc + 1.0))
    off = jnp.concatenate(offs, axis=1)
    xo1 = x1 + off
    yo1 = y1 + off
    xo2 = x2 + off
    yo2 = y2 + off
    areas = (xo2 - xo1) * (yo2 - yo1)

    def body(k, carry):
        sc, out = carry
        kf = k.astype(jnp.float32)
        sc_new = []
        out_new = []
        for b in range(nimg):
            sl = sls[b]
            scb = sc[:, sl]
            m = jnp.max(scb)
            ok = m > 0.0
            selidx = jnp.min(jnp.where(scb == m, idxf, 1e9))
            sel = (idxf == selidx) & ok

            def pick(a):
                return jnp.sum(jnp.where(sel, a, 0.0))

            bx1 = pick(xo1[:, sl])
            by1 = pick(yo1[:, sl])
            bx2 = pick(xo2[:, sl])
            by2 = pick(yo2[:, sl])
            boff = pick(off[:, sl])
            barea = (bx2 - bx1) * (by2 - by1)
            xx1 = jnp.maximum(bx1, xo1[:, sl])
            yy1 = jnp.maximum(by1, yo1[:, sl])
            xx2 = jnp.minimum(bx2, xo2[:, sl])
            yy2 = jnp.minimum(by2, yo2[:, sl])
            inter = (jnp.maximum(xx2 - xx1, 0.0)
                     * jnp.maximum(yy2 - yy1, 0.0))
            iou = inter / (barea + areas[:, sl] - inter + 1e-9)
            kill = (iou > NMS_THRESH) | sel
            sc_new.append(jnp.where(kill & ok, -1.0, scb))

            lab = jnp.floor(boff / (maxcs[b] + 1.0) + 0.5)
            vals = [bx1 - boff, by1 - boff, bx2 - boff, by2 - boff, m, lab]
            slot = (lane128 == kf) & ok
            ob = out[:, sl]
            for j, v in enumerate(vals):
                ob = jnp.where(slot & (row8 == float(j)), v, ob)
            out_new.append(ob)
        return jnp.concatenate(sc_new, axis=1), jnp.concatenate(out_new,
                                                                axis=1)

    out0 = jnp.zeros((8, 128 * nimg), jnp.float32)
    sc, out = jax.lax.fori_loop(0, DETS_PER_IMG, body, (sc0, out0))

    for b in range(nimg):
        sl = slice(128 * b, 128 * b + DETS_PER_IMG)
        bt = jnp.transpose(out[0:4, sls[b]])
        boxes_ref[b] = bt[0:DETS_PER_IMG, :]
        scores_ref[b] = out[4:5, sl]
        labout_ref[b] = out[5:6, sl].astype(jnp.int32)


@jax.jit
def kernel(class_logits, bbox_deltas, roi_proposals, resized_image_sizes):
    B, N, C = class_logits.shape

    RB = 1000
    nrb = N // RB
    s1 = pl.pallas_call(
        _stage1_kernel,
        grid=(B, nrb),
        in_specs=[
            pl.BlockSpec((1, RB, C), lambda b, rb: (b, rb, 0)),
            pl.BlockSpec((1, RB, 4 * C), lambda b, rb: (b, rb, 0)),
            pl.BlockSpec((1, RB, 4), lambda b, rb: (b, rb, 0)),
            pl.BlockSpec((1, 1, 2), lambda b, rb: (b, 0, 0)),
        ],
        out_specs=[pl.BlockSpec((1, RB, 1), lambda b, rb: (b, rb, 0))] * 6,
        out_shape=[jax.ShapeDtypeStruct((B, N, 1), jnp.float32)] * 6,
    )(class_logits, bbox_deltas, roi_proposals,
      resized_image_sizes.reshape(B, 1, 2))
    probs, labels, x1, y1, x2, y2 = s1

    def prep(a, padval):
        a = a.reshape(B, N)
        a = jnp.pad(a, ((0, 0), (0, N_PAD - N)), constant_values=padval)
        a = a.reshape(B, N_PAD // 128, 128)
        return jnp.transpose(a, (1, 0, 2)).reshape(N_PAD // 128, B * 128)

    ins = [prep(probs, -1.0), prep(labels, 0.0), prep(x1, 0.0),
           prep(y1, 0.0), prep(x2, 0.0), prep(y2, 0.0)]

    R = N_PAD // 128
    boxes, scores, labout = pl.pallas_call(
        functools.partial(_stage2_kernel, B),
        in_specs=[pl.BlockSpec((R, B * 128), lambda: (0, 0))] * 6,
        out_specs=[
            pl.BlockSpec((B, DETS_PER_IMG, 4), lambda: (0, 0, 0)),
            pl.BlockSpec((B, 1, DETS_PER_IMG), lambda: (0, 0, 0)),
            pl.BlockSpec((B, 1, DETS_PER_IMG), lambda: (0, 0, 0)),
        ],
        out_shape=[
            jax.ShapeDtypeStruct((B, DETS_PER_IMG, 4), jnp.float32),
            jax.ShapeDtypeStruct((B, 1, DETS_PER_IMG), jnp.float32),
            jax.ShapeDtypeStruct((B, 1, DETS_PER_IMG), jnp.int32),
        ],
    )(*ins)

    return (boxes, scores.reshape(B, DETS_PER_IMG),
            labout.reshape(B, DETS_PER_IMG))
